# speedup vs baseline: 1.0481x; 1.0481x over previous
import functools

import jax
import jax.numpy as jnp
from jax.experimental import pallas as pl
from jax.experimental.pallas import tpu as pltpu

N = 10000
BM = 1000


def _pool_body(h_ref, w_ref, b_ref, o_ref):
    o_ref[...] = jax.nn.relu(
        jnp.dot(h_ref[...], w_ref[...], preferred_element_type=jnp.float32)
        + b_ref[...]
    )


def _pool_matmul(h, Wp, bp):
    din = h.shape[1]
    return pl.pallas_call(
        _pool_body,
        grid=(N // BM,),
        in_specs=[
            pl.BlockSpec((BM, din), lambda i: (i, 0)),
            pl.BlockSpec((din, din), lambda i: (0, 0)),
            pl.BlockSpec((din,), lambda i: (0,)),
        ],
        out_specs=pl.BlockSpec((BM, din), lambda i: (i, 0)),
        out_shape=jax.ShapeDtypeStruct((N, din), jnp.float32),
    )(h, Wp, bp)


def _combine_body(act, h_ref, hn_ref, ws_ref, wn_ref, b_ref, o_ref):
    r = (
        jnp.dot(h_ref[...], ws_ref[...], preferred_element_type=jnp.float32)
        + jnp.dot(hn_ref[...], wn_ref[...], preferred_element_type=jnp.float32)
        + b_ref[...]
    )
    if act:
        r = jax.nn.relu(r)
    o_ref[...] = r


def _combine_matmul(h, hn, Ws, Wn, b, act):
    din, dout = Ws.shape
    return pl.pallas_call(
        functools.partial(_combine_body, act),
        grid=(N // BM,),
        in_specs=[
            pl.BlockSpec((BM, din), lambda i: (i, 0)),
            pl.BlockSpec((BM, din), lambda i: (i, 0)),
            pl.BlockSpec((din, dout), lambda i: (0, 0)),
            pl.BlockSpec((din, dout), lambda i: (0, 0)),
            pl.BlockSpec((dout,), lambda i: (0,)),
        ],
        out_specs=pl.BlockSpec((BM, dout), lambda i: (i, 0)),
        out_shape=jax.ShapeDtypeStruct((N, dout), jnp.float32),
    )(h, hn, Ws, Wn, b)


def _segment_max(feat_src, src, dst):
    msg = jnp.take(feat_src, src, axis=0)
    hn = jax.ops.segment_max(msg, dst, num_segments=N)
    return jnp.where(jnp.isfinite(hn), hn, 0.0)


def _layer(h, src, dst, Wp, bp, Ws, Wn, b, act):
    fs = _pool_matmul(h, Wp, bp)
    hn = _segment_max(fs, src, dst)
    return _combine_matmul(h, hn, Ws, Wn, b, act)


def kernel(features, edge_index, Wp1, bp1, Ws1, Wn1, b1, Wp2, bp2, Ws2, Wn2,
           b2, Wp3, bp3, Ws3, Wn3, b3):
    src = edge_index[0].astype(jnp.int32)
    dst = edge_index[1].astype(jnp.int32)
    h1 = _layer(features, src, dst, Wp1, bp1, Ws1, Wn1, b1, act=True)
    aspect = _layer(h1, src, dst, Wp2, bp2, Ws2, Wn2, b2, act=False)
    out = _layer(aspect, src, dst, Wp3, bp3, Ws3, Wn3, b3, act=False)
    return (aspect, out)


# speedup vs baseline: 1.4430x; 1.3767x over previous
import dataclasses
import functools

import jax
import jax.numpy as jnp
from jax import lax
from jax.experimental import pallas as pl
from jax.experimental.pallas import tpu as pltpu
from jax.experimental.pallas import tpu_sc as plsc

N = 10000
BM = 1000

NW = 32
RANGE = 320
NPAD = NW * RANGE
E = 320000
EC = 4000
G = 128


def _pool_body(h_ref, w_ref, b_ref, o_ref):
    o_ref[...] = jax.nn.relu(
        jnp.dot(h_ref[...], w_ref[...], preferred_element_type=jnp.float32)
        + b_ref[...]
    )


def _pool_matmul(h, Wp, bp):
    din = h.shape[1]
    return pl.pallas_call(
        _pool_body,
        grid=(N // BM,),
        in_specs=[
            pl.BlockSpec((BM, din), lambda i: (i, 0)),
            pl.BlockSpec((din, din), lambda i: (0, 0)),
            pl.BlockSpec((din,), lambda i: (0,)),
        ],
        out_specs=pl.BlockSpec((BM, din), lambda i: (i, 0)),
        out_shape=jax.ShapeDtypeStruct((N, din), jnp.float32),
    )(h, Wp, bp)


def _combine_body(act, h_ref, hn_ref, ws_ref, wn_ref, b_ref, o_ref):
    r = (
        jnp.dot(h_ref[...], ws_ref[...], preferred_element_type=jnp.float32)
        + jnp.dot(hn_ref[...], wn_ref[...], preferred_element_type=jnp.float32)
        + b_ref[...]
    )
    if act:
        r = jax.nn.relu(r)
    o_ref[...] = r


def _combine_matmul(h, hn, Ws, Wn, b, act):
    din, dout = Ws.shape
    return pl.pallas_call(
        functools.partial(_combine_body, act),
        grid=(N // BM,),
        in_specs=[
            pl.BlockSpec((BM, din), lambda i: (i, 0)),
            pl.BlockSpec((BM, din), lambda i: (i, 0)),
            pl.BlockSpec((din, dout), lambda i: (0, 0)),
            pl.BlockSpec((din, dout), lambda i: (0, 0)),
            pl.BlockSpec((dout,), lambda i: (0,)),
        ],
        out_specs=pl.BlockSpec((BM, dout), lambda i: (i, 0)),
        out_shape=jax.ShapeDtypeStruct((N, dout), jnp.float32),
    )(h, hn, Ws, Wn, b)


def _segmax_body(D, src_hbm, dst_hbm, feat_hbm, out_hbm, src_c, dst_c,
                 sel_s, sel_d, rows, acc, cnt_ref):
    NCH = D // 16
    wid = lax.axis_index("s") * 2 + lax.axis_index("c")
    lo = wid * RANGE
    hi = lo + RANGE

    cnt_ref[0] = 0

    @pl.loop(0, RANGE)
    def _(r):
        for j in range(NCH):
            acc[r, pl.ds(j * 16, 16)] = jnp.zeros((16,), jnp.float32)

    for k in range(G // 16 + 1):
        sel_s[pl.ds(k * 16, 16)] = jnp.zeros((16,), jnp.int32)
        sel_d[pl.ds(k * 16, 16)] = jnp.zeros((16,), jnp.int32)

    def flush(nblocks):
        pltpu.sync_copy(feat_hbm.at[sel_s.at[pl.ds(0, G)]], rows)

        @pl.loop(0, nblocks)
        def _(b):
            bo = pl.multiple_of(b * 16, 16)
            dl16 = sel_d[pl.ds(bo, 16)]
            for l in range(16):
                dl = dl16[l]
                i = bo + l
                for j in range(NCH):
                    sl = pl.ds(j * 16, 16)
                    acc[dl, sl] = jnp.maximum(acc[dl, sl], rows[i, sl])

    @pl.loop(0, E // EC)
    def _(c):
        off = pl.multiple_of(c * EC, EC)
        pltpu.sync_copy(src_hbm.at[pl.ds(off, EC)], src_c)
        pltpu.sync_copy(dst_hbm.at[pl.ds(off, EC)], dst_c)

        @pl.loop(0, EC // 16)
        def _(g):
            bo = pl.multiple_of(g * 16, 16)
            d16 = dst_c[pl.ds(bo, 16)]
            s16 = src_c[pl.ds(bo, 16)]
            m = (d16 >= lo) & (d16 < hi)
            cnt = cnt_ref[0]
            plsc.store_compressed(sel_s.at[pl.ds(cnt, 16)], s16, mask=m)
            plsc.store_compressed(sel_d.at[pl.ds(cnt, 16)], d16 - lo, mask=m)
            newcnt = cnt + jnp.sum(m.astype(jnp.int32))
            cnt_ref[0] = newcnt

            @pl.when(newcnt >= G)
            def _():
                flush(G // 16)
                srem = sel_s[pl.ds(G, 16)]
                drem = sel_d[pl.ds(G, 16)]
                sel_s[pl.ds(0, 16)] = srem
                sel_d[pl.ds(0, 16)] = drem
                cnt_ref[0] = newcnt - G

    fincnt = cnt_ref[0]

    @pl.when(fincnt > 0)
    def _():
        sel_s[pl.ds(fincnt, 16)] = jnp.zeros((16,), jnp.int32)
        sel_d[pl.ds(fincnt, 16)] = jnp.full((16,), RANGE, jnp.int32)
        flush((fincnt + 15) // 16)

    pltpu.sync_copy(acc.at[pl.ds(0, RANGE)], out_hbm.at[pl.ds(lo, RANGE)])


@functools.lru_cache(maxsize=None)
def _make_segmax(D):
    mesh = plsc.VectorSubcoreMesh(core_axis_name="c", subcore_axis_name="s")
    cp = pltpu.CompilerParams()
    if "needs_layout_passes" in pltpu.CompilerParams.__dataclass_fields__:
        cp = dataclasses.replace(cp, needs_layout_passes=False)
    return pl.kernel(
        functools.partial(_segmax_body, D),
        out_type=jax.ShapeDtypeStruct((NPAD, D), jnp.float32),
        mesh=mesh,
        compiler_params=cp,
        scratch_types=[
            pltpu.VMEM((EC,), jnp.int32),
            pltpu.VMEM((EC,), jnp.int32),
            pltpu.VMEM((G + 16,), jnp.int32),
            pltpu.VMEM((G + 16,), jnp.int32),
            pltpu.VMEM((G, D), jnp.float32),
            pltpu.VMEM((RANGE + 1, D), jnp.float32),
            pltpu.SMEM((1,), jnp.int32),
        ],
    )


def _segment_max(feat_src, src, dst):
    D = feat_src.shape[1]
    if D < 128:
        feat_src = jnp.pad(feat_src, ((0, 0), (0, 128 - D)))
    out = _make_segmax(128)(src, dst, feat_src)
    return out[:N, :D]


def _layer(h, src, dst, Wp, bp, Ws, Wn, b, act):
    fs = _pool_matmul(h, Wp, bp)
    hn = _segment_max(fs, src, dst)
    return _combine_matmul(h, hn, Ws, Wn, b, act)


def kernel(features, edge_index, Wp1, bp1, Ws1, Wn1, b1, Wp2, bp2, Ws2, Wn2,
           b2, Wp3, bp3, Ws3, Wn3, b3):
    src = edge_index[0].astype(jnp.int32)
    dst = edge_index[1].astype(jnp.int32)
    h1 = _layer(features, src, dst, Wp1, bp1, Ws1, Wn1, b1, act=True)
    aspect = _layer(h1, src, dst, Wp2, bp2, Ws2, Wn2, b2, act=False)
    out = _layer(aspect, src, dst, Wp3, bp3, Ws3, Wn3, b3, act=False)
    return (aspect, out)
